# baseline (speedup 1.0000x reference)
import functools

import jax
import jax.numpy as jnp
from jax import lax
from jax.experimental import pallas as pl
from jax.experimental.pallas import tpu as pltpu
from jax.experimental.pallas import tpu_sc as plsc

IN_CH = 96
OUT_CH = 192
KK = 3
HW = 224
TABLE_SIZE = 16
RHASH = 0.05
RB = 8


def _stats_body(x_ref, a_ref, o_ref):
    b = pl.program_id(0)
    xb = x_ref[...]
    av = a_ref[...]
    neg = jnp.float32(-3.0e38)
    ri = lax.broadcasted_iota(jnp.int32, (8, HW, HW), 1)
    cm = [
        jnp.max(jnp.where(ri <= HW - 2, xb, neg), axis=1),
        jnp.max(xb, axis=1),
        jnp.max(jnp.where(ri >= 1, xb, neg), axis=1),
    ]
    ci = lax.broadcasted_iota(jnp.int32, (8, HW), 1)
    cmask = [ci <= HW - 2, ci >= 0, ci >= 1]
    ms = []
    for kh in range(3):
        for kw in range(3):
            m = jnp.max(jnp.where(cmask[kw], cm[kh], neg), axis=1)
            if not (kh == 1 and kw == 1):
                m = jnp.maximum(m, 0.0)
            ms.append(m)
    mr = jnp.stack(ms, axis=1)
    pdot = jnp.sum(mr * av[:, :9])
    pnrm = jnp.sum(mr * mr)

    @pl.when(b == 0)
    def _():
        o_ref[...] = jnp.zeros_like(o_ref)

    li = lax.broadcasted_iota(jnp.int32, (8, 128), 1)
    si = lax.broadcasted_iota(jnp.int32, (8, 128), 0)
    contrib = jnp.where((si == 0) & (li == 1), pdot, 0.0) + jnp.where(
        (si == 0) & (li == 2), pnrm, 0.0)
    o_ref[...] += contrib


def _stats_call(xs, a9p):
    return pl.pallas_call(
        _stats_body,
        grid=(IN_CH // 8,),
        in_specs=[
            pl.BlockSpec((8, HW, HW), lambda b: (b, 0, 0)),
            pl.BlockSpec((8, 16), lambda b: (b, 0)),
        ],
        out_specs=pl.BlockSpec((8, 128), lambda b: (0, 0)),
        out_shape=jax.ShapeDtypeStruct((8, 128), jnp.float32),
    )(xs, a9p)


def _route_body(stats_hbm, par_hbm, offs_hbm, tidx_hbm, s_hbm,
                st_v, par_v, offs_v, tidx_v, s_v):
    cid = lax.axis_index("c")
    sid = lax.axis_index("s")

    @pl.when((cid == 0) & (sid == 0))
    def _():
        pltpu.sync_copy(stats_hbm, st_v)
        pltpu.sync_copy(par_hbm, par_v)
        pltpu.sync_copy(offs_hbm, offs_v)
        pltpu.sync_copy(tidx_hbm, tidx_v)
        lanes = lax.iota(jnp.int32, 16)
        zero_i = jnp.zeros((16,), jnp.int32)
        dt = plsc.load_gather(st_v, [zero_i + 1])
        nr = plsc.load_gather(st_v, [zero_i + 2])
        tail = plsc.load_gather(par_v, [zero_i + 1])
        nb = plsc.bitcast(nr, jnp.int32)
        y = plsc.bitcast((nb >> 1) + jnp.int32(0x1FBD1DF5), jnp.float32)
        for _ in range(5):
            y = 0.5 * (y + nr / y)
        hval = (dt / y + tail) / RHASH
        ti = hval.astype(jnp.int32)
        tif = ti.astype(jnp.float32)
        ti = jnp.where(tif > hval, ti - 1, ti)
        idx = jnp.where(ti < 0, -ti, ti) % TABLE_SIZE
        start = plsc.load_gather(offs_v, [idx])
        end = plsc.load_gather(offs_v, [idx + 1])
        count = end - start
        cf = jnp.where(count > 0, count, 1).astype(jnp.float32)
        scale = jnp.where(count > 0, float(OUT_CH) / cf, 1.0)
        init = jnp.where(count > 0, 0.0, 1.0)
        for c in range(OUT_CH // 16):
            s_v[pl.ds(16 * c, 16)] = init
        for c in range(OUT_CH // 16):
            pos = lanes + 16 * c
            m = (pos >= start) & (pos < end)
            tix = tidx_v[pl.ds(16 * c, 16)]
            plsc.store_scatter(s_v, [tix], scale, mask=m)
        pltpu.sync_copy(s_v, s_hbm)


def _route_call(stats16, params16, offs32, tidx32):
    mesh = plsc.VectorSubcoreMesh(core_axis_name="c", subcore_axis_name="s")
    f = functools.partial(
        pl.kernel,
        out_type=jax.ShapeDtypeStruct((OUT_CH,), jnp.float32),
        mesh=mesh,
        compiler_params=pltpu.CompilerParams(needs_layout_passes=False),
        scratch_types=[
            pltpu.VMEM((16,), jnp.float32),
            pltpu.VMEM((16,), jnp.float32),
            pltpu.VMEM((32,), jnp.int32),
            pltpu.VMEM((OUT_CH,), jnp.int32),
            pltpu.VMEM((OUT_CH,), jnp.float32),
        ],
    )(_route_body)
    return f(stats16, params16, offs32, tidx32)


def _repack_body(x_hbm, xt_hbm, zrow, sem):
    zrow[...] = jnp.zeros_like(zrow)
    cps = [
        pltpu.make_async_copy(zrow, xt_hbm.at[0], sem),
        pltpu.make_async_copy(zrow, xt_hbm.at[HW + 1], sem),
    ]
    for c in range(IN_CH):
        cps.append(pltpu.make_async_copy(
            x_hbm.at[c], xt_hbm.at[pl.ds(1, HW), c], sem))
    for cp in cps:
        cp.start()
    for cp in cps:
        cp.wait()


def _repack_call(xs):
    return pl.pallas_call(
        _repack_body,
        in_specs=[pl.BlockSpec(memory_space=pl.ANY)],
        out_specs=pl.BlockSpec(memory_space=pl.ANY),
        out_shape=jax.ShapeDtypeStruct((HW + 2, IN_CH, HW), jnp.float32),
        scratch_shapes=[
            pltpu.VMEM((IN_CH, HW), jnp.float32),
            pltpu.SemaphoreType.DMA,
        ],
    )(xs)


def _conv_body(s_ref, b_ref, w_ref, xt_hbm, out_ref, xs_raw, xp_ref, sem):
    g = pl.program_id(0)
    cp = pltpu.make_async_copy(
        xt_hbm.at[pl.ds(g * RB, RB + 2)], xs_raw, sem)
    cp.start()

    @pl.when(g == 0)
    def _():
        xp_ref[:, :, 0:1] = jnp.zeros((RB + 2, IN_CH, 1), jnp.float32)
        xp_ref[:, :, HW + 1:HW + 2] = jnp.zeros((RB + 2, IN_CH, 1),
                                                jnp.float32)

    cp.wait()
    xp_ref[:, :, 1:HW + 1] = xs_raw[...]
    sv = s_ref[...]
    bv = b_ref[...]
    for i in range(RB):
        acc = jnp.zeros((OUT_CH, HW), jnp.float32)
        for kh in range(3):
            xrow = xp_ref[i + kh]
            for kw in range(3):
                y = lax.dot_general(
                    w_ref[3 * kh + kw], xrow,
                    dimension_numbers=(((1,), (0,)), ((), ())),
                    preferred_element_type=jnp.float32)
                acc = acc + y[:, kw:kw + HW]
        out_ref[:, i, :] = acc * sv + bv


def _conv_call(svec, bias_col, w9, xt):
    return pl.pallas_call(
        _conv_body,
        grid=(HW // RB,),
        in_specs=[
            pl.BlockSpec((OUT_CH, 1), lambda g: (0, 0)),
            pl.BlockSpec((OUT_CH, 1), lambda g: (0, 0)),
            pl.BlockSpec((9, OUT_CH, IN_CH), lambda g: (0, 0, 0)),
            pl.BlockSpec(memory_space=pl.ANY),
        ],
        out_specs=pl.BlockSpec((OUT_CH, RB, HW), lambda g: (0, g, 0)),
        out_shape=jax.ShapeDtypeStruct((OUT_CH, HW, HW), jnp.float32),
        scratch_shapes=[
            pltpu.VMEM((RB + 2, IN_CH, HW), jnp.float32),
            pltpu.VMEM((RB + 2, IN_CH, HW + 2), jnp.float32),
            pltpu.SemaphoreType.DMA,
        ],
    )(svec, bias_col, w9, xt)


def kernel(x, mode, kernels, bias, a_hash, b_hash, table_indices,
           table_offsets):
    del mode
    f32 = jnp.float32
    xs = x.reshape(IN_CH, HW, HW).astype(f32)
    w9 = jnp.transpose(
        kernels.reshape(OUT_CH, IN_CH, KK, KK), (2, 3, 0, 1)
    ).reshape(KK * KK, OUT_CH, IN_CH)
    a9p = jnp.zeros((IN_CH, 16), f32).at[:, :9].set(
        a_hash[:IN_CH * 9].reshape(IN_CH, 9))
    tail_c = 0.5 * jnp.sum(a_hash[IN_CH * 9:IN_CH * 9 + 3]) + b_hash
    params = jnp.zeros((16,), f32).at[1].set(tail_c)
    offs32 = jnp.zeros((32,), jnp.int32).at[:TABLE_SIZE + 1].set(
        table_offsets.astype(jnp.int32))
    tidx32 = table_indices.astype(jnp.int32)

    xt = _repack_call(xs)
    stats = _stats_call(xs, a9p)
    svec = _route_call(stats[0, :16], params, offs32, tidx32)
    out = _conv_call(svec.reshape(OUT_CH, 1), bias.reshape(OUT_CH, 1),
                     w9, xt)
    return out.reshape(1, OUT_CH, HW, HW)


# speedup vs baseline: 4.0392x; 4.0392x over previous
import functools

import jax
import jax.numpy as jnp
from jax import lax
from jax.experimental import pallas as pl
from jax.experimental.pallas import tpu as pltpu
from jax.experimental.pallas import tpu_sc as plsc

IN_CH = 96
OUT_CH = 192
KK = 3
HW = 224
TABLE_SIZE = 16
RHASH = 0.05
RB = 8


def _stats_body(x_ref, a_ref, o_ref):
    b = pl.program_id(0)
    xb = x_ref[...]
    av = a_ref[...]
    neg = jnp.float32(-3.0e38)
    ri = lax.broadcasted_iota(jnp.int32, (8, HW, HW), 1)
    cm = [
        jnp.max(jnp.where(ri <= HW - 2, xb, neg), axis=1),
        jnp.max(xb, axis=1),
        jnp.max(jnp.where(ri >= 1, xb, neg), axis=1),
    ]
    ci = lax.broadcasted_iota(jnp.int32, (8, HW), 1)
    cmask = [ci <= HW - 2, ci >= 0, ci >= 1]
    ms = []
    for kh in range(3):
        for kw in range(3):
            m = jnp.max(jnp.where(cmask[kw], cm[kh], neg), axis=1)
            if not (kh == 1 and kw == 1):
                m = jnp.maximum(m, 0.0)
            ms.append(m)
    mr = jnp.stack(ms, axis=1)
    pdot = jnp.sum(mr * av[:, :9])
    pnrm = jnp.sum(mr * mr)

    @pl.when(b == 0)
    def _():
        o_ref[...] = jnp.zeros_like(o_ref)

    li = lax.broadcasted_iota(jnp.int32, (8, 128), 1)
    si = lax.broadcasted_iota(jnp.int32, (8, 128), 0)
    contrib = jnp.where((si == 0) & (li == 1), pdot, 0.0) + jnp.where(
        (si == 0) & (li == 2), pnrm, 0.0)
    o_ref[...] += contrib


def _stats_call(xs, a9p):
    return pl.pallas_call(
        _stats_body,
        grid=(IN_CH // 8,),
        in_specs=[
            pl.BlockSpec((8, HW, HW), lambda b: (b, 0, 0)),
            pl.BlockSpec((8, 16), lambda b: (b, 0)),
        ],
        out_specs=pl.BlockSpec((8, 128), lambda b: (0, 0)),
        out_shape=jax.ShapeDtypeStruct((8, 128), jnp.float32),
    )(xs, a9p)


def _route_body(stats_hbm, par_hbm, offs_hbm, tidx_hbm, s_hbm,
                st_v, par_v, offs_v, tidx_v, s_v):
    cid = lax.axis_index("c")
    sid = lax.axis_index("s")

    @pl.when((cid == 0) & (sid == 0))
    def _():
        pltpu.sync_copy(stats_hbm, st_v)
        pltpu.sync_copy(par_hbm, par_v)
        pltpu.sync_copy(offs_hbm, offs_v)
        pltpu.sync_copy(tidx_hbm, tidx_v)
        lanes = lax.iota(jnp.int32, 16)
        zero_i = jnp.zeros((16,), jnp.int32)
        dt = plsc.load_gather(st_v, [zero_i + 1])
        nr = plsc.load_gather(st_v, [zero_i + 2])
        tail = plsc.load_gather(par_v, [zero_i + 1])
        nb = plsc.bitcast(nr, jnp.int32)
        y = plsc.bitcast((nb >> 1) + jnp.int32(0x1FBD1DF5), jnp.float32)
        for _ in range(5):
            y = 0.5 * (y + nr / y)
        hval = (dt / y + tail) / RHASH
        ti = hval.astype(jnp.int32)
        tif = ti.astype(jnp.float32)
        ti = jnp.where(tif > hval, ti - 1, ti)
        idx = jnp.where(ti < 0, -ti, ti) % TABLE_SIZE
        start = plsc.load_gather(offs_v, [idx])
        end = plsc.load_gather(offs_v, [idx + 1])
        count = end - start
        cf = jnp.where(count > 0, count, 1).astype(jnp.float32)
        scale = jnp.where(count > 0, float(OUT_CH) / cf, 1.0)
        init = jnp.where(count > 0, 0.0, 1.0)
        for c in range(OUT_CH // 16):
            s_v[pl.ds(16 * c, 16)] = init
        for c in range(OUT_CH // 16):
            pos = lanes + 16 * c
            m = (pos >= start) & (pos < end)
            tix = tidx_v[pl.ds(16 * c, 16)]
            plsc.store_scatter(s_v, [tix], scale, mask=m)
        pltpu.sync_copy(s_v, s_hbm)


def _route_call(stats16, params16, offs32, tidx32):
    mesh = plsc.VectorSubcoreMesh(core_axis_name="c", subcore_axis_name="s")
    f = functools.partial(
        pl.kernel,
        out_type=jax.ShapeDtypeStruct((OUT_CH,), jnp.float32),
        mesh=mesh,
        compiler_params=pltpu.CompilerParams(needs_layout_passes=False),
        scratch_types=[
            pltpu.VMEM((16,), jnp.float32),
            pltpu.VMEM((16,), jnp.float32),
            pltpu.VMEM((32,), jnp.int32),
            pltpu.VMEM((OUT_CH,), jnp.int32),
            pltpu.VMEM((OUT_CH,), jnp.float32),
        ],
    )(_route_body)
    return f(stats16, params16, offs32, tidx32)


def _conv_body(s_ref, b_ref, w_ref, xt_hbm, out_ref, xs_raw, xp_ref, sem):
    g = pl.program_id(0)
    ng = HW // RB
    zrow = jnp.zeros((1, IN_CH, HW), jnp.float32)

    @pl.when(g == 0)
    def _():
        cp = pltpu.make_async_copy(
            xt_hbm.at[pl.ds(0, RB + 1)], xs_raw.at[pl.ds(1, RB + 1)], sem)
        cp.start()
        xs_raw[0:1] = zrow
        cp.wait()

    @pl.when(g == ng - 1)
    def _():
        cp = pltpu.make_async_copy(
            xt_hbm.at[pl.ds(HW - RB - 1, RB + 1)],
            xs_raw.at[pl.ds(0, RB + 1)], sem)
        cp.start()
        xs_raw[RB + 1:RB + 2] = zrow
        cp.wait()

    @pl.when((g > 0) & (g < ng - 1))
    def _():
        cp = pltpu.make_async_copy(
            xt_hbm.at[pl.ds(g * RB - 1, RB + 2)], xs_raw, sem)
        cp.start()
        cp.wait()

    @pl.when(g == 0)
    def _():
        xp_ref[:, :, 0:1] = jnp.zeros((RB + 2, IN_CH, 1), jnp.float32)
        xp_ref[:, :, HW + 1:HW + 2] = jnp.zeros((RB + 2, IN_CH, 1),
                                                jnp.float32)
    xp_ref[:, :, 1:HW + 1] = xs_raw[...]
    sv = s_ref[...]
    bv = b_ref[...]
    for i in range(RB):
        acc = jnp.zeros((OUT_CH, HW), jnp.float32)
        for kh in range(3):
            xrow = xp_ref[i + kh]
            for kw in range(3):
                y = lax.dot_general(
                    w_ref[3 * kh + kw], xrow,
                    dimension_numbers=(((1,), (0,)), ((), ())),
                    preferred_element_type=jnp.float32)
                acc = acc + y[:, kw:kw + HW]
        out_ref[:, i, :] = acc * sv + bv


def _conv_call(svec, bias_col, w9, xt):
    return pl.pallas_call(
        _conv_body,
        grid=(HW // RB,),
        in_specs=[
            pl.BlockSpec((OUT_CH, 1), lambda g: (0, 0)),
            pl.BlockSpec((OUT_CH, 1), lambda g: (0, 0)),
            pl.BlockSpec((9, OUT_CH, IN_CH), lambda g: (0, 0, 0)),
            pl.BlockSpec(memory_space=pl.ANY),
        ],
        out_specs=pl.BlockSpec((OUT_CH, RB, HW), lambda g: (0, g, 0)),
        out_shape=jax.ShapeDtypeStruct((OUT_CH, HW, HW), jnp.float32),
        scratch_shapes=[
            pltpu.VMEM((RB + 2, IN_CH, HW), jnp.float32),
            pltpu.VMEM((RB + 2, IN_CH, HW + 2), jnp.float32),
            pltpu.SemaphoreType.DMA,
        ],
    )(svec, bias_col, w9, xt)


def kernel(x, mode, kernels, bias, a_hash, b_hash, table_indices,
           table_offsets):
    del mode
    f32 = jnp.float32
    xs = x.reshape(IN_CH, HW, HW).astype(f32)
    w9 = jnp.transpose(
        kernels.reshape(OUT_CH, IN_CH, KK, KK), (2, 3, 0, 1)
    ).reshape(KK * KK, OUT_CH, IN_CH)
    a9p = jnp.zeros((IN_CH, 16), f32).at[:, :9].set(
        a_hash[:IN_CH * 9].reshape(IN_CH, 9))
    tail_c = 0.5 * jnp.sum(a_hash[IN_CH * 9:IN_CH * 9 + 3]) + b_hash
    params = jnp.zeros((16,), f32).at[1].set(tail_c)
    offs32 = jnp.zeros((32,), jnp.int32).at[:TABLE_SIZE + 1].set(
        table_offsets.astype(jnp.int32))
    tidx32 = table_indices.astype(jnp.int32)

    xt = jnp.transpose(xs, (1, 0, 2))
    stats = _stats_call(xs, a9p)
    svec = _route_call(stats[0, :16], params, offs32, tidx32)
    out = _conv_call(svec.reshape(OUT_CH, 1), bias.reshape(OUT_CH, 1),
                     w9, xt)
    return out.reshape(1, OUT_CH, HW, HW)


# speedup vs baseline: 4.3035x; 1.0654x over previous
import functools

import jax
import jax.numpy as jnp
from jax import lax
from jax.experimental import pallas as pl
from jax.experimental.pallas import tpu as pltpu
from jax.experimental.pallas import tpu_sc as plsc

IN_CH = 96
OUT_CH = 192
KK = 3
HW = 224
TABLE_SIZE = 16
RHASH = 0.05
RB = 8


def _stats_body(x_ref, a_ref, o_ref, xt_ref):
    b = pl.program_id(0)
    xb = x_ref[...]
    xt_ref[...] = jnp.transpose(xb, (1, 0, 2))
    av = a_ref[...]
    neg = jnp.float32(-3.0e38)
    ri = lax.broadcasted_iota(jnp.int32, (8, HW, HW), 1)
    cm = [
        jnp.max(jnp.where(ri <= HW - 2, xb, neg), axis=1),
        jnp.max(xb, axis=1),
        jnp.max(jnp.where(ri >= 1, xb, neg), axis=1),
    ]
    ci = lax.broadcasted_iota(jnp.int32, (8, HW), 1)
    cmask = [ci <= HW - 2, ci >= 0, ci >= 1]
    ms = []
    for kh in range(3):
        for kw in range(3):
            m = jnp.max(jnp.where(cmask[kw], cm[kh], neg), axis=1)
            if not (kh == 1 and kw == 1):
                m = jnp.maximum(m, 0.0)
            ms.append(m)
    mr = jnp.stack(ms, axis=1)
    pdot = jnp.sum(mr * av[:, :9])
    pnrm = jnp.sum(mr * mr)

    @pl.when(b == 0)
    def _():
        o_ref[...] = jnp.zeros_like(o_ref)

    li = lax.broadcasted_iota(jnp.int32, (8, 128), 1)
    si = lax.broadcasted_iota(jnp.int32, (8, 128), 0)
    contrib = jnp.where((si == 0) & (li == 1), pdot, 0.0) + jnp.where(
        (si == 0) & (li == 2), pnrm, 0.0)
    o_ref[...] += contrib


def _stats_call(xs, a9p):
    return pl.pallas_call(
        _stats_body,
        grid=(IN_CH // 8,),
        in_specs=[
            pl.BlockSpec((8, HW, HW), lambda b: (b, 0, 0)),
            pl.BlockSpec((8, 16), lambda b: (b, 0)),
        ],
        out_specs=[
            pl.BlockSpec((8, 128), lambda b: (0, 0)),
            pl.BlockSpec((HW, 8, HW), lambda b: (0, b, 0)),
        ],
        out_shape=[
            jax.ShapeDtypeStruct((8, 128), jnp.float32),
            jax.ShapeDtypeStruct((HW, IN_CH, HW), jnp.float32),
        ],
    )(xs, a9p)


def _route_body(stats_hbm, par_hbm, offs_hbm, tidx_hbm, s_hbm,
                st_v, par_v, offs_v, tidx_v, s_v):
    cid = lax.axis_index("c")
    sid = lax.axis_index("s")

    @pl.when((cid == 0) & (sid == 0))
    def _():
        pltpu.sync_copy(stats_hbm, st_v)
        pltpu.sync_copy(par_hbm, par_v)
        pltpu.sync_copy(offs_hbm, offs_v)
        pltpu.sync_copy(tidx_hbm, tidx_v)
        lanes = lax.iota(jnp.int32, 16)
        zero_i = jnp.zeros((16,), jnp.int32)
        dt = plsc.load_gather(st_v, [zero_i + 1])
        nr = plsc.load_gather(st_v, [zero_i + 2])
        tail = plsc.load_gather(par_v, [zero_i + 1])
        nb = plsc.bitcast(nr, jnp.int32)
        y = plsc.bitcast((nb >> 1) + jnp.int32(0x1FBD1DF5), jnp.float32)
        for _ in range(5):
            y = 0.5 * (y + nr / y)
        hval = (dt / y + tail) / RHASH
        ti = hval.astype(jnp.int32)
        tif = ti.astype(jnp.float32)
        ti = jnp.where(tif > hval, ti - 1, ti)
        idx = jnp.where(ti < 0, -ti, ti) % TABLE_SIZE
        start = plsc.load_gather(offs_v, [idx])
        end = plsc.load_gather(offs_v, [idx + 1])
        count = end - start
        cf = jnp.where(count > 0, count, 1).astype(jnp.float32)
        scale = jnp.where(count > 0, float(OUT_CH) / cf, 1.0)
        init = jnp.where(count > 0, 0.0, 1.0)
        for c in range(OUT_CH // 16):
            s_v[pl.ds(16 * c, 16)] = init
        for c in range(OUT_CH // 16):
            pos = lanes + 16 * c
            m = (pos >= start) & (pos < end)
            tix = tidx_v[pl.ds(16 * c, 16)]
            plsc.store_scatter(s_v, [tix], scale, mask=m)
        pltpu.sync_copy(s_v, s_hbm)


def _route_call(stats16, params16, offs32, tidx32):
    mesh = plsc.VectorSubcoreMesh(core_axis_name="c", subcore_axis_name="s")
    f = functools.partial(
        pl.kernel,
        out_type=jax.ShapeDtypeStruct((OUT_CH,), jnp.float32),
        mesh=mesh,
        compiler_params=pltpu.CompilerParams(needs_layout_passes=False),
        scratch_types=[
            pltpu.VMEM((16,), jnp.float32),
            pltpu.VMEM((16,), jnp.float32),
            pltpu.VMEM((32,), jnp.int32),
            pltpu.VMEM((OUT_CH,), jnp.int32),
            pltpu.VMEM((OUT_CH,), jnp.float32),
        ],
    )(_route_body)
    return f(stats16, params16, offs32, tidx32)


def _conv_body(s_ref, b_ref, w_ref, xt_hbm, out_ref, xs_raw, xp_ref, sem):
    g = pl.program_id(0)
    ng = HW // RB
    zrow = jnp.zeros((1, IN_CH, HW), jnp.float32)

    @pl.when(g == 0)
    def _():
        cp = pltpu.make_async_copy(
            xt_hbm.at[pl.ds(0, RB + 1)], xs_raw.at[pl.ds(1, RB + 1)], sem)
        cp.start()
        xs_raw[0:1] = zrow
        cp.wait()

    @pl.when(g == ng - 1)
    def _():
        cp = pltpu.make_async_copy(
            xt_hbm.at[pl.ds(HW - RB - 1, RB + 1)],
            xs_raw.at[pl.ds(0, RB + 1)], sem)
        cp.start()
        xs_raw[RB + 1:RB + 2] = zrow
        cp.wait()

    @pl.when((g > 0) & (g < ng - 1))
    def _():
        cp = pltpu.make_async_copy(
            xt_hbm.at[pl.ds(g * RB - 1, RB + 2)], xs_raw, sem)
        cp.start()
        cp.wait()

    @pl.when(g == 0)
    def _():
        xp_ref[:, :, 0:1] = jnp.zeros((RB + 2, IN_CH, 1), jnp.float32)
        xp_ref[:, :, HW + 1:HW + 2] = jnp.zeros((RB + 2, IN_CH, 1),
                                                jnp.float32)
    xp_ref[:, :, 1:HW + 1] = xs_raw[...]
    sv = s_ref[...]
    bv = b_ref[...]
    for i in range(RB):
        acc = jnp.zeros((OUT_CH, HW), jnp.float32)
        for kh in range(3):
            xrow = xp_ref[i + kh]
            for kw in range(3):
                y = lax.dot_general(
                    w_ref[3 * kh + kw], xrow,
                    dimension_numbers=(((1,), (0,)), ((), ())),
                    preferred_element_type=jnp.float32)
                acc = acc + y[:, kw:kw + HW]
        out_ref[:, i, :] = acc * sv + bv


def _conv_call(svec, bias_col, w9, xt):
    return pl.pallas_call(
        _conv_body,
        grid=(HW // RB,),
        in_specs=[
            pl.BlockSpec((OUT_CH, 1), lambda g: (0, 0)),
            pl.BlockSpec((OUT_CH, 1), lambda g: (0, 0)),
            pl.BlockSpec((9, OUT_CH, IN_CH), lambda g: (0, 0, 0)),
            pl.BlockSpec(memory_space=pl.ANY),
        ],
        out_specs=pl.BlockSpec((OUT_CH, RB, HW), lambda g: (0, g, 0)),
        out_shape=jax.ShapeDtypeStruct((OUT_CH, HW, HW), jnp.float32),
        scratch_shapes=[
            pltpu.VMEM((RB + 2, IN_CH, HW), jnp.float32),
            pltpu.VMEM((RB + 2, IN_CH, HW + 2), jnp.float32),
            pltpu.SemaphoreType.DMA,
        ],
    )(svec, bias_col, w9, xt)


def kernel(x, mode, kernels, bias, a_hash, b_hash, table_indices,
           table_offsets):
    del mode
    f32 = jnp.float32
    xs = x.reshape(IN_CH, HW, HW).astype(f32)
    w9 = jnp.transpose(
        kernels.reshape(OUT_CH, IN_CH, KK, KK), (2, 3, 0, 1)
    ).reshape(KK * KK, OUT_CH, IN_CH)
    a9p = jnp.zeros((IN_CH, 16), f32).at[:, :9].set(
        a_hash[:IN_CH * 9].reshape(IN_CH, 9))
    tail_c = 0.5 * jnp.sum(a_hash[IN_CH * 9:IN_CH * 9 + 3]) + b_hash
    params = jnp.zeros((16,), f32).at[1].set(tail_c)
    offs32 = jnp.zeros((32,), jnp.int32).at[:TABLE_SIZE + 1].set(
        table_offsets.astype(jnp.int32))
    tidx32 = table_indices.astype(jnp.int32)

    stats, xt = _stats_call(xs, a9p)
    svec = _route_call(stats[0, :16], params, offs32, tidx32)
    out = _conv_call(svec.reshape(OUT_CH, 1), bias.reshape(OUT_CH, 1),
                     w9, xt)
    return out.reshape(1, OUT_CH, HW, HW)


# speedup vs baseline: 4.4039x; 1.0233x over previous
import functools

import jax
import jax.numpy as jnp
from jax import lax
from jax.experimental import pallas as pl
from jax.experimental.pallas import tpu as pltpu
from jax.experimental.pallas import tpu_sc as plsc

IN_CH = 96
OUT_CH = 192
KK = 3
HW = 224
TABLE_SIZE = 16
RHASH = 0.05
RB = 8


def _stats_body(x_ref, a_ref, o_ref, xt_ref):
    b = pl.program_id(0)
    xb = x_ref[...]
    xt_ref[...] = jnp.pad(jnp.transpose(xb, (1, 0, 2)),
                          ((1, 1), (0, 0), (1, 1)))
    av = a_ref[...]
    neg = jnp.float32(-3.0e38)
    ri = lax.broadcasted_iota(jnp.int32, (8, HW, HW), 1)
    cm = [
        jnp.max(jnp.where(ri <= HW - 2, xb, neg), axis=1),
        jnp.max(xb, axis=1),
        jnp.max(jnp.where(ri >= 1, xb, neg), axis=1),
    ]
    ci = lax.broadcasted_iota(jnp.int32, (8, HW), 1)
    cmask = [ci <= HW - 2, ci >= 0, ci >= 1]
    ms = []
    for kh in range(3):
        for kw in range(3):
            m = jnp.max(jnp.where(cmask[kw], cm[kh], neg), axis=1)
            if not (kh == 1 and kw == 1):
                m = jnp.maximum(m, 0.0)
            ms.append(m)
    mr = jnp.stack(ms, axis=1)
    pdot = jnp.sum(mr * av[:, :9])
    pnrm = jnp.sum(mr * mr)

    @pl.when(b == 0)
    def _():
        o_ref[...] = jnp.zeros_like(o_ref)

    li = lax.broadcasted_iota(jnp.int32, (8, 128), 1)
    si = lax.broadcasted_iota(jnp.int32, (8, 128), 0)
    contrib = jnp.where((si == 0) & (li == 1), pdot, 0.0) + jnp.where(
        (si == 0) & (li == 2), pnrm, 0.0)
    o_ref[...] += contrib


def _stats_call(xs, a9p):
    return pl.pallas_call(
        _stats_body,
        grid=(IN_CH // 8,),
        in_specs=[
            pl.BlockSpec((8, HW, HW), lambda b: (b, 0, 0)),
            pl.BlockSpec((8, 16), lambda b: (b, 0)),
        ],
        out_specs=[
            pl.BlockSpec((8, 128), lambda b: (0, 0)),
            pl.BlockSpec((HW + 2, 8, HW + 2), lambda b: (0, b, 0)),
        ],
        out_shape=[
            jax.ShapeDtypeStruct((8, 128), jnp.float32),
            jax.ShapeDtypeStruct((HW + 2, IN_CH, HW + 2), jnp.float32),
        ],
    )(xs, a9p)


def _route_body(stats_hbm, par_hbm, offs_hbm, tidx_hbm, s_hbm,
                st_v, par_v, offs_v, tidx_v, s_v):
    cid = lax.axis_index("c")
    sid = lax.axis_index("s")

    @pl.when((cid == 0) & (sid == 0))
    def _():
        pltpu.sync_copy(stats_hbm, st_v)
        pltpu.sync_copy(par_hbm, par_v)
        pltpu.sync_copy(offs_hbm, offs_v)
        pltpu.sync_copy(tidx_hbm, tidx_v)
        lanes = lax.iota(jnp.int32, 16)
        zero_i = jnp.zeros((16,), jnp.int32)
        dt = plsc.load_gather(st_v, [zero_i + 1])
        nr = plsc.load_gather(st_v, [zero_i + 2])
        tail = plsc.load_gather(par_v, [zero_i + 1])
        nb = plsc.bitcast(nr, jnp.int32)
        y = plsc.bitcast((nb >> 1) + jnp.int32(0x1FBD1DF5), jnp.float32)
        for _ in range(5):
            y = 0.5 * (y + nr / y)
        hval = (dt / y + tail) / RHASH
        ti = hval.astype(jnp.int32)
        tif = ti.astype(jnp.float32)
        ti = jnp.where(tif > hval, ti - 1, ti)
        idx = jnp.where(ti < 0, -ti, ti) % TABLE_SIZE
        start = plsc.load_gather(offs_v, [idx])
        end = plsc.load_gather(offs_v, [idx + 1])
        count = end - start
        cf = jnp.where(count > 0, count, 1).astype(jnp.float32)
        scale = jnp.where(count > 0, float(OUT_CH) / cf, 1.0)
        init = jnp.where(count > 0, 0.0, 1.0)
        for c in range(OUT_CH // 16):
            s_v[pl.ds(16 * c, 16)] = init
        for c in range(OUT_CH // 16):
            pos = lanes + 16 * c
            m = (pos >= start) & (pos < end)
            tix = tidx_v[pl.ds(16 * c, 16)]
            plsc.store_scatter(s_v, [tix], scale, mask=m)
        pltpu.sync_copy(s_v, s_hbm)


def _route_call(stats16, params16, offs32, tidx32):
    mesh = plsc.VectorSubcoreMesh(core_axis_name="c", subcore_axis_name="s")
    f = functools.partial(
        pl.kernel,
        out_type=jax.ShapeDtypeStruct((OUT_CH,), jnp.float32),
        mesh=mesh,
        compiler_params=pltpu.CompilerParams(needs_layout_passes=False),
        scratch_types=[
            pltpu.VMEM((16,), jnp.float32),
            pltpu.VMEM((16,), jnp.float32),
            pltpu.VMEM((32,), jnp.int32),
            pltpu.VMEM((OUT_CH,), jnp.int32),
            pltpu.VMEM((OUT_CH,), jnp.float32),
        ],
    )(_route_body)
    return f(stats16, params16, offs32, tidx32)


def _conv_body(s_ref, b_ref, w_ref, xt_hbm, out_ref, xp_ref, sem):
    g = pl.program_id(0)
    cp = pltpu.make_async_copy(
        xt_hbm.at[pl.ds(g * RB, RB + 2)], xp_ref, sem)
    cp.start()
    cp.wait()
    sv = s_ref[...]
    bv = b_ref[...]
    for i in range(RB):
        acc = jnp.zeros((OUT_CH, HW), jnp.float32)
        for kh in range(3):
            xrow = xp_ref[i + kh]
            for kw in range(3):
                y = lax.dot_general(
                    w_ref[3 * kh + kw], xrow,
                    dimension_numbers=(((1,), (0,)), ((), ())),
                    preferred_element_type=jnp.float32)
                acc = acc + y[:, kw:kw + HW]
        out_ref[:, i, :] = acc * sv + bv


def _conv_call(svec, bias_col, w9, xt):
    return pl.pallas_call(
        _conv_body,
        grid=(HW // RB,),
        in_specs=[
            pl.BlockSpec((OUT_CH, 1), lambda g: (0, 0)),
            pl.BlockSpec((OUT_CH, 1), lambda g: (0, 0)),
            pl.BlockSpec((9, OUT_CH, IN_CH), lambda g: (0, 0, 0)),
            pl.BlockSpec(memory_space=pl.ANY),
        ],
        out_specs=pl.BlockSpec((OUT_CH, RB, HW), lambda g: (0, g, 0)),
        out_shape=jax.ShapeDtypeStruct((OUT_CH, HW, HW), jnp.float32),
        scratch_shapes=[
            pltpu.VMEM((RB + 2, IN_CH, HW + 2), jnp.float32),
            pltpu.SemaphoreType.DMA,
        ],
    )(svec, bias_col, w9, xt)


def kernel(x, mode, kernels, bias, a_hash, b_hash, table_indices,
           table_offsets):
    del mode
    f32 = jnp.float32
    xs = x.reshape(IN_CH, HW, HW).astype(f32)
    w9 = jnp.transpose(
        kernels.reshape(OUT_CH, IN_CH, KK, KK), (2, 3, 0, 1)
    ).reshape(KK * KK, OUT_CH, IN_CH)
    a9p = jnp.zeros((IN_CH, 16), f32).at[:, :9].set(
        a_hash[:IN_CH * 9].reshape(IN_CH, 9))
    tail_c = 0.5 * jnp.sum(a_hash[IN_CH * 9:IN_CH * 9 + 3]) + b_hash
    params = jnp.zeros((16,), f32).at[1].set(tail_c)
    offs32 = jnp.zeros((32,), jnp.int32).at[:TABLE_SIZE + 1].set(
        table_offsets.astype(jnp.int32))
    tidx32 = table_indices.astype(jnp.int32)

    stats, xt = _stats_call(xs, a9p)
    svec = _route_call(stats[0, :16], params, offs32, tidx32)
    out = _conv_call(svec.reshape(OUT_CH, 1), bias.reshape(OUT_CH, 1),
                     w9, xt)
    return out.reshape(1, OUT_CH, HW, HW)


# speedup vs baseline: 4.5080x; 1.0236x over previous
import functools

import jax
import jax.numpy as jnp
from jax import lax
from jax.experimental import pallas as pl
from jax.experimental.pallas import tpu as pltpu
from jax.experimental.pallas import tpu_sc as plsc

IN_CH = 96
OUT_CH = 192
KK = 3
HW = 224
TABLE_SIZE = 16
RHASH = 0.05
RB = 16


def _stats_body(x_ref, a_ref, o_ref, xt_ref):
    b = pl.program_id(0)
    xb = x_ref[...]
    xt_ref[...] = jnp.pad(jnp.transpose(xb, (1, 0, 2)),
                          ((1, 1), (0, 0), (1, 1)))
    av = a_ref[...]
    neg = jnp.float32(-3.0e38)
    ri = lax.broadcasted_iota(jnp.int32, (8, HW, HW), 1)
    cm = [
        jnp.max(jnp.where(ri <= HW - 2, xb, neg), axis=1),
        jnp.max(xb, axis=1),
        jnp.max(jnp.where(ri >= 1, xb, neg), axis=1),
    ]
    ci = lax.broadcasted_iota(jnp.int32, (8, HW), 1)
    cmask = [ci <= HW - 2, ci >= 0, ci >= 1]
    ms = []
    for kh in range(3):
        for kw in range(3):
            m = jnp.max(jnp.where(cmask[kw], cm[kh], neg), axis=1)
            if not (kh == 1 and kw == 1):
                m = jnp.maximum(m, 0.0)
            ms.append(m)
    mr = jnp.stack(ms, axis=1)
    pdot = jnp.sum(mr * av[:, :9])
    pnrm = jnp.sum(mr * mr)

    @pl.when(b == 0)
    def _():
        o_ref[...] = jnp.zeros_like(o_ref)

    li = lax.broadcasted_iota(jnp.int32, (8, 128), 1)
    si = lax.broadcasted_iota(jnp.int32, (8, 128), 0)
    contrib = jnp.where((si == 0) & (li == 1), pdot, 0.0) + jnp.where(
        (si == 0) & (li == 2), pnrm, 0.0)
    o_ref[...] += contrib


def _stats_call(xs, a9p):
    return pl.pallas_call(
        _stats_body,
        grid=(IN_CH // 8,),
        in_specs=[
            pl.BlockSpec((8, HW, HW), lambda b: (b, 0, 0)),
            pl.BlockSpec((8, 16), lambda b: (b, 0)),
        ],
        out_specs=[
            pl.BlockSpec((8, 128), lambda b: (0, 0)),
            pl.BlockSpec((HW + 2, 8, HW + 2), lambda b: (0, b, 0)),
        ],
        out_shape=[
            jax.ShapeDtypeStruct((8, 128), jnp.float32),
            jax.ShapeDtypeStruct((HW + 2, IN_CH, HW + 2), jnp.float32),
        ],
    )(xs, a9p)


def _route_body(stats_hbm, par_hbm, offs_hbm, tidx_hbm, s_hbm,
                st_v, par_v, offs_v, tidx_v, s_v):
    cid = lax.axis_index("c")
    sid = lax.axis_index("s")

    @pl.when((cid == 0) & (sid == 0))
    def _():
        pltpu.sync_copy(stats_hbm, st_v)
        pltpu.sync_copy(par_hbm, par_v)
        pltpu.sync_copy(offs_hbm, offs_v)
        pltpu.sync_copy(tidx_hbm, tidx_v)
        lanes = lax.iota(jnp.int32, 16)
        zero_i = jnp.zeros((16,), jnp.int32)
        dt = plsc.load_gather(st_v, [zero_i + 1])
        nr = plsc.load_gather(st_v, [zero_i + 2])
        tail = plsc.load_gather(par_v, [zero_i + 1])
        nb = plsc.bitcast(nr, jnp.int32)
        y = plsc.bitcast((nb >> 1) + jnp.int32(0x1FBD1DF5), jnp.float32)
        for _ in range(5):
            y = 0.5 * (y + nr / y)
        hval = (dt / y + tail) / RHASH
        ti = hval.astype(jnp.int32)
        tif = ti.astype(jnp.float32)
        ti = jnp.where(tif > hval, ti - 1, ti)
        idx = jnp.where(ti < 0, -ti, ti) % TABLE_SIZE
        start = plsc.load_gather(offs_v, [idx])
        end = plsc.load_gather(offs_v, [idx + 1])
        count = end - start
        cf = jnp.where(count > 0, count, 1).astype(jnp.float32)
        scale = jnp.where(count > 0, float(OUT_CH) / cf, 1.0)
        init = jnp.where(count > 0, 0.0, 1.0)
        for c in range(OUT_CH // 16):
            s_v[pl.ds(16 * c, 16)] = init
        for c in range(OUT_CH // 16):
            pos = lanes + 16 * c
            m = (pos >= start) & (pos < end)
            tix = tidx_v[pl.ds(16 * c, 16)]
            plsc.store_scatter(s_v, [tix], scale, mask=m)
        pltpu.sync_copy(s_v, s_hbm)


def _route_call(stats16, params16, offs32, tidx32):
    mesh = plsc.VectorSubcoreMesh(core_axis_name="c", subcore_axis_name="s")
    f = functools.partial(
        pl.kernel,
        out_type=jax.ShapeDtypeStruct((OUT_CH,), jnp.float32),
        mesh=mesh,
        compiler_params=pltpu.CompilerParams(needs_layout_passes=False),
        scratch_types=[
            pltpu.VMEM((16,), jnp.float32),
            pltpu.VMEM((16,), jnp.float32),
            pltpu.VMEM((32,), jnp.int32),
            pltpu.VMEM((OUT_CH,), jnp.int32),
            pltpu.VMEM((OUT_CH,), jnp.float32),
        ],
    )(_route_body)
    return f(stats16, params16, offs32, tidx32)


def _conv_body(s_ref, b_ref, w_ref, xt_hbm, out_ref, xp_ref, sem):
    g = pl.program_id(0)
    cp = pltpu.make_async_copy(
        xt_hbm.at[pl.ds(g * RB, RB + 2)], xp_ref, sem)
    cp.start()
    cp.wait()
    sv = s_ref[...]
    bv = b_ref[...]
    for i in range(RB):
        acc = jnp.zeros((OUT_CH, HW), jnp.float32)
        for kh in range(3):
            xrow = xp_ref[i + kh]
            for kw in range(3):
                y = lax.dot_general(
                    w_ref[3 * kh + kw], xrow,
                    dimension_numbers=(((1,), (0,)), ((), ())),
                    preferred_element_type=jnp.float32)
                acc = acc + y[:, kw:kw + HW]
        out_ref[:, i, :] = acc * sv + bv


def _conv_call(svec, bias_col, w9, xt):
    return pl.pallas_call(
        _conv_body,
        grid=(HW // RB,),
        in_specs=[
            pl.BlockSpec((OUT_CH, 1), lambda g: (0, 0)),
            pl.BlockSpec((OUT_CH, 1), lambda g: (0, 0)),
            pl.BlockSpec((9, OUT_CH, IN_CH), lambda g: (0, 0, 0)),
            pl.BlockSpec(memory_space=pl.ANY),
        ],
        out_specs=pl.BlockSpec((OUT_CH, RB, HW), lambda g: (0, g, 0)),
        out_shape=jax.ShapeDtypeStruct((OUT_CH, HW, HW), jnp.float32),
        scratch_shapes=[
            pltpu.VMEM((RB + 2, IN_CH, HW + 2), jnp.float32),
            pltpu.SemaphoreType.DMA,
        ],
    )(svec, bias_col, w9, xt)


def kernel(x, mode, kernels, bias, a_hash, b_hash, table_indices,
           table_offsets):
    del mode
    f32 = jnp.float32
    xs = x.reshape(IN_CH, HW, HW).astype(f32)
    w9 = jnp.transpose(
        kernels.reshape(OUT_CH, IN_CH, KK, KK), (2, 3, 0, 1)
    ).reshape(KK * KK, OUT_CH, IN_CH)
    a9p = jnp.zeros((IN_CH, 16), f32).at[:, :9].set(
        a_hash[:IN_CH * 9].reshape(IN_CH, 9))
    tail_c = 0.5 * jnp.sum(a_hash[IN_CH * 9:IN_CH * 9 + 3]) + b_hash
    params = jnp.zeros((16,), f32).at[1].set(tail_c)
    offs32 = jnp.zeros((32,), jnp.int32).at[:TABLE_SIZE + 1].set(
        table_offsets.astype(jnp.int32))
    tidx32 = table_indices.astype(jnp.int32)

    stats, xt = _stats_call(xs, a9p)
    svec = _route_call(stats[0, :16], params, offs32, tidx32)
    out = _conv_call(svec.reshape(OUT_CH, 1), bias.reshape(OUT_CH, 1),
                     w9, xt)
    return out.reshape(1, OUT_CH, HW, HW)


# speedup vs baseline: 5.9870x; 1.3281x over previous
import functools

import jax
import jax.numpy as jnp
from jax import lax
from jax.experimental import pallas as pl
from jax.experimental.pallas import tpu as pltpu
from jax.experimental.pallas import tpu_sc as plsc

IN_CH = 96
OUT_CH = 192
KK = 3
HW = 224
TABLE_SIZE = 16
RHASH = 0.05
RB = 8


def _stats_body(x_ref, a_ref, o_ref, xt_ref):
    b = pl.program_id(0)
    xb = x_ref[...]
    xt_ref[...] = jnp.pad(jnp.transpose(xb, (1, 0, 2)),
                          ((1, 1), (0, 0), (1, 1)))
    av = a_ref[...]
    neg = jnp.float32(-3.0e38)
    ri = lax.broadcasted_iota(jnp.int32, (8, HW, HW), 1)
    cm = [
        jnp.max(jnp.where(ri <= HW - 2, xb, neg), axis=1),
        jnp.max(xb, axis=1),
        jnp.max(jnp.where(ri >= 1, xb, neg), axis=1),
    ]
    ci = lax.broadcasted_iota(jnp.int32, (8, HW), 1)
    cmask = [ci <= HW - 2, ci >= 0, ci >= 1]
    ms = []
    for kh in range(3):
        for kw in range(3):
            m = jnp.max(jnp.where(cmask[kw], cm[kh], neg), axis=1)
            if not (kh == 1 and kw == 1):
                m = jnp.maximum(m, 0.0)
            ms.append(m)
    mr = jnp.stack(ms, axis=1)
    pdot = jnp.sum(mr * av[:, :9])
    pnrm = jnp.sum(mr * mr)

    @pl.when(b == 0)
    def _():
        o_ref[...] = jnp.zeros_like(o_ref)

    li = lax.broadcasted_iota(jnp.int32, (8, 128), 1)
    si = lax.broadcasted_iota(jnp.int32, (8, 128), 0)
    contrib = jnp.where((si == 0) & (li == 1), pdot, 0.0) + jnp.where(
        (si == 0) & (li == 2), pnrm, 0.0)
    o_ref[...] += contrib


def _stats_call(xs, a9p):
    return pl.pallas_call(
        _stats_body,
        grid=(IN_CH // 8,),
        in_specs=[
            pl.BlockSpec((8, HW, HW), lambda b: (b, 0, 0)),
            pl.BlockSpec((8, 16), lambda b: (b, 0)),
        ],
        out_specs=[
            pl.BlockSpec((8, 128), lambda b: (0, 0)),
            pl.BlockSpec((HW + 2, 8, HW + 2), lambda b: (0, b, 0)),
        ],
        out_shape=[
            jax.ShapeDtypeStruct((8, 128), jnp.float32),
            jax.ShapeDtypeStruct((HW + 2, IN_CH, HW + 2), jnp.float32),
        ],
    )(xs, a9p)


def _route_body(stats_hbm, par_hbm, offs_hbm, tidx_hbm, s_hbm,
                st_v, par_v, offs_v, tidx_v, s_v):
    cid = lax.axis_index("c")
    sid = lax.axis_index("s")

    @pl.when((cid == 0) & (sid == 0))
    def _():
        pltpu.sync_copy(stats_hbm, st_v)
        pltpu.sync_copy(par_hbm, par_v)
        pltpu.sync_copy(offs_hbm, offs_v)
        pltpu.sync_copy(tidx_hbm, tidx_v)
        lanes = lax.iota(jnp.int32, 16)
        zero_i = jnp.zeros((16,), jnp.int32)
        dt = plsc.load_gather(st_v, [zero_i + 1])
        nr = plsc.load_gather(st_v, [zero_i + 2])
        tail = plsc.load_gather(par_v, [zero_i + 1])
        nb = plsc.bitcast(nr, jnp.int32)
        y = plsc.bitcast((nb >> 1) + jnp.int32(0x1FBD1DF5), jnp.float32)
        for _ in range(5):
            y = 0.5 * (y + nr / y)
        hval = (dt / y + tail) / RHASH
        ti = hval.astype(jnp.int32)
        tif = ti.astype(jnp.float32)
        ti = jnp.where(tif > hval, ti - 1, ti)
        idx = jnp.where(ti < 0, -ti, ti) % TABLE_SIZE
        start = plsc.load_gather(offs_v, [idx])
        end = plsc.load_gather(offs_v, [idx + 1])
        count = end - start
        cf = jnp.where(count > 0, count, 1).astype(jnp.float32)
        scale = jnp.where(count > 0, float(OUT_CH) / cf, 1.0)
        init = jnp.where(count > 0, 0.0, 1.0)
        for c in range(OUT_CH // 16):
            s_v[pl.ds(16 * c, 16)] = init
        for c in range(OUT_CH // 16):
            pos = lanes + 16 * c
            m = (pos >= start) & (pos < end)
            tix = tidx_v[pl.ds(16 * c, 16)]
            plsc.store_scatter(s_v, [tix], scale, mask=m)
        pltpu.sync_copy(s_v, s_hbm)


def _route_call(stats16, params16, offs32, tidx32):
    mesh = plsc.VectorSubcoreMesh(core_axis_name="c", subcore_axis_name="s")
    f = functools.partial(
        pl.kernel,
        out_type=jax.ShapeDtypeStruct((OUT_CH,), jnp.float32),
        mesh=mesh,
        compiler_params=pltpu.CompilerParams(needs_layout_passes=False),
        scratch_types=[
            pltpu.VMEM((16,), jnp.float32),
            pltpu.VMEM((16,), jnp.float32),
            pltpu.VMEM((32,), jnp.int32),
            pltpu.VMEM((OUT_CH,), jnp.int32),
            pltpu.VMEM((OUT_CH,), jnp.float32),
        ],
    )(_route_body)
    return f(stats16, params16, offs32, tidx32)


def _conv_body(s_ref, b_ref, w_ref, xt_hbm, out_ref, xp_ref, sem):
    g = pl.program_id(0)
    ng = HW // RB
    slot = lax.rem(g, 2)
    nslot = lax.rem(g + 1, 2)

    @pl.when(g == 0)
    def _():
        pltpu.make_async_copy(
            xt_hbm.at[pl.ds(0, RB + 2)], xp_ref.at[0], sem.at[0]).start()

    @pl.when(g + 1 < ng)
    def _():
        pltpu.make_async_copy(
            xt_hbm.at[pl.ds((g + 1) * RB, RB + 2)], xp_ref.at[nslot],
            sem.at[nslot]).start()

    pltpu.make_async_copy(
        xt_hbm.at[pl.ds(g * RB, RB + 2)], xp_ref.at[slot],
        sem.at[slot]).wait()
    sv = s_ref[...]
    bv = b_ref[...]
    for i in range(RB):
        acc = jnp.zeros((OUT_CH, HW), jnp.float32)
        for kh in range(3):
            xrow = xp_ref[slot, i + kh]
            for kw in range(3):
                y = lax.dot_general(
                    w_ref[3 * kh + kw], xrow,
                    dimension_numbers=(((1,), (0,)), ((), ())),
                    preferred_element_type=jnp.float32)
                acc = acc + y[:, kw:kw + HW]
        out_ref[:, i, :] = acc * sv + bv


def _conv_call(svec, bias_col, w9, xt):
    return pl.pallas_call(
        _conv_body,
        grid=(HW // RB,),
        in_specs=[
            pl.BlockSpec((OUT_CH, 1), lambda g: (0, 0)),
            pl.BlockSpec((OUT_CH, 1), lambda g: (0, 0)),
            pl.BlockSpec((9, OUT_CH, IN_CH), lambda g: (0, 0, 0)),
            pl.BlockSpec(memory_space=pl.ANY),
        ],
        out_specs=pl.BlockSpec((OUT_CH, RB, HW), lambda g: (0, g, 0)),
        out_shape=jax.ShapeDtypeStruct((OUT_CH, HW, HW), jnp.float32),
        scratch_shapes=[
            pltpu.VMEM((2, RB + 2, IN_CH, HW + 2), jnp.float32),
            pltpu.SemaphoreType.DMA((2,)),
        ],
    )(svec, bias_col, w9, xt)


def kernel(x, mode, kernels, bias, a_hash, b_hash, table_indices,
           table_offsets):
    del mode
    f32 = jnp.float32
    xs = x.reshape(IN_CH, HW, HW).astype(f32)
    w9 = jnp.transpose(
        kernels.reshape(OUT_CH, IN_CH, KK, KK), (2, 3, 0, 1)
    ).reshape(KK * KK, OUT_CH, IN_CH)
    a9p = jnp.zeros((IN_CH, 16), f32).at[:, :9].set(
        a_hash[:IN_CH * 9].reshape(IN_CH, 9))
    tail_c = 0.5 * jnp.sum(a_hash[IN_CH * 9:IN_CH * 9 + 3]) + b_hash
    params = jnp.zeros((16,), f32).at[1].set(tail_c)
    offs32 = jnp.zeros((32,), jnp.int32).at[:TABLE_SIZE + 1].set(
        table_offsets.astype(jnp.int32))
    tidx32 = table_indices.astype(jnp.int32)

    stats, xt = _stats_call(xs, a9p)
    svec = _route_call(stats[0, :16], params, offs32, tidx32)
    out = _conv_call(svec.reshape(OUT_CH, 1), bias.reshape(OUT_CH, 1),
                     w9, xt)
    return out.reshape(1, OUT_CH, HW, HW)
